# Initial kernel scaffold; baseline (speedup 1.0000x reference)
import jax
import jax.numpy as jnp
from jax.experimental import pallas as pl


def kernel(x, table, W, b):
    raise NotImplementedError("write your pallas kernel here")


# speedup vs baseline: 2.7215x; 2.7215x over previous
import functools

import jax
import jax.numpy as jnp
from jax import lax
from jax.experimental import pallas as pl
from jax.experimental.pallas import tpu as pltpu
from jax.experimental.pallas import tpu_sc as plsc

B = 16384
L = 50
E = 32
OUT = 64

NC = 2
NS = 16
NW = NC * NS

RW = B // NW
CR = 32
NCH = RW // CR
NI = CR * L
G = 80
NG = NI // G


def _sc_pool_kernel(x_hbm, table_hbm, out_hbm, idx_v, rows_v, pooled_v, sem):
    wid = lax.axis_index("s") * NC + lax.axis_index("c")
    base_row = wid * RW

    def chunk_body(g, carry):
        row0 = base_row + g * CR
        pltpu.sync_copy(x_hbm.at[pl.ds(row0 * L, NI)], idx_v)
        copies = []
        for j in range(NG):
            copies.append(
                pltpu.async_copy(
                    table_hbm.at[idx_v.at[pl.ds(j * G, G)]],
                    rows_v.at[pl.ds(j * G, G)],
                    sem,
                )
            )
        for c in copies:
            c.wait()

        def reduce_body(c, carry2):
            r0 = c * L
            acc0 = rows_v[r0, pl.ds(0, 16)]
            acc1 = rows_v[r0, pl.ds(16, 16)]
            for l in range(1, L):
                acc0 = acc0 + rows_v[r0 + l, pl.ds(0, 16)]
                acc1 = acc1 + rows_v[r0 + l, pl.ds(16, 16)]
            pooled_v[c, pl.ds(0, 16)] = acc0
            pooled_v[c, pl.ds(16, 16)] = acc1
            return carry2

        lax.fori_loop(0, CR, reduce_body, 0, unroll=False)
        pltpu.sync_copy(pooled_v, out_hbm.at[pl.ds(row0, CR)])
        return carry

    lax.fori_loop(0, NCH, chunk_body, 0, unroll=False)


@jax.jit
def _sc_pool(x_flat, table):
    mesh = plsc.VectorSubcoreMesh(core_axis_name="c", subcore_axis_name="s")
    kfn = pl.kernel(
        _sc_pool_kernel,
        mesh=mesh,
        out_type=jax.ShapeDtypeStruct((B, E), jnp.float32),
        scratch_types=[
            pltpu.VMEM((NI,), jnp.int32),
            pltpu.VMEM((NI, E), jnp.float32),
            pltpu.VMEM((CR, E), jnp.float32),
            pltpu.SemaphoreType.DMA,
        ],
        compiler_params=pltpu.CompilerParams(use_tc_tiling_on_sc=False),
    )
    return kfn(x_flat, table)


def _tc_matmul_kernel(p_ref, wt_ref, b_ref, o_ref):
    o_ref[...] = (
        jnp.dot(p_ref[...], wt_ref[...], preferred_element_type=jnp.float32)
        + b_ref[...]
    )


@jax.jit
def _tc_matmul(pooled, Wt, b2):
    BLK = 2048
    return pl.pallas_call(
        _tc_matmul_kernel,
        grid=(B // BLK,),
        in_specs=[
            pl.BlockSpec((BLK, E), lambda i: (i, 0)),
            pl.BlockSpec((E, OUT), lambda i: (0, 0)),
            pl.BlockSpec((1, OUT), lambda i: (0, 0)),
        ],
        out_specs=pl.BlockSpec((BLK, OUT), lambda i: (i, 0)),
        out_shape=jax.ShapeDtypeStruct((B, OUT), jnp.float32),
    )(pooled, Wt, b2)


def kernel(x, table, W, b):
    x_flat = x.astype(jnp.int32).reshape(-1)
    pooled = _sc_pool(x_flat, table)
    return _tc_matmul(pooled, W.T, b.reshape(1, OUT))


# speedup vs baseline: 4.1941x; 1.5411x over previous
import functools

import jax
import jax.numpy as jnp
from jax import lax
from jax.experimental import pallas as pl
from jax.experimental.pallas import tpu as pltpu
from jax.experimental.pallas import tpu_sc as plsc

B = 16384
L = 50
E = 32
OUT = 64

NC = 2
NS = 16
NW = NC * NS

RW = B // NW
CR = 32
NCH = RW // CR
NI = CR * L
G = 80
NG = NI // G


def _sc_pool_kernel(x_hbm, table_hbm, out_hbm, idx_v, rows_v, pooled_v, sem):
    wid = lax.axis_index("s") * NC + lax.axis_index("c")
    base_row = wid * RW

    def chunk_body(g, carry):
        row0 = base_row + g * CR
        pltpu.sync_copy(x_hbm.at[pl.ds(row0 * L, NI)], idx_v)
        copies = []
        for j in range(NG):
            copies.append(
                pltpu.async_copy(
                    table_hbm.at[idx_v.at[pl.ds(j * G, G)]],
                    rows_v.at[pl.ds(j * G, G)],
                    sem,
                )
            )
        for c in copies:
            c.wait()

        def reduce_body(c, carry2):
            r0 = c * L
            acc0 = rows_v[r0, pl.ds(0, 16)]
            acc1 = rows_v[r0, pl.ds(16, 16)]
            for l in range(1, L):
                acc0 = acc0 + rows_v[r0 + l, pl.ds(0, 16)]
                acc1 = acc1 + rows_v[r0 + l, pl.ds(16, 16)]
            pooled_v[c, pl.ds(0, 16)] = acc0
            pooled_v[c, pl.ds(16, 16)] = acc1
            return carry2

        lax.fori_loop(0, CR, reduce_body, 0, unroll=False)
        pltpu.sync_copy(pooled_v, out_hbm.at[pl.ds(row0, CR)])
        return carry

    lax.fori_loop(0, NCH, chunk_body, 0, unroll=False)


@jax.jit
def _sc_pool(x_flat, table):
    mesh = plsc.VectorSubcoreMesh(core_axis_name="c", subcore_axis_name="s")
    kfn = pl.kernel(
        _sc_pool_kernel,
        mesh=mesh,
        out_type=jax.ShapeDtypeStruct((B, E), jnp.float32),
        scratch_types=[
            pltpu.VMEM((NI,), jnp.int32),
            pltpu.VMEM((NI, E), jnp.float32),
            pltpu.VMEM((CR, E), jnp.float32),
            pltpu.SemaphoreType.DMA,
        ],
        compiler_params=pltpu.CompilerParams(use_tc_tiling_on_sc=False),
    )
    return kfn(x_flat, table)


VOCAB = 1000000
NB = 2048
NBLK = 123
QPAD = NB * NBLK
VPAD = 4 * QPAD
LASTBLK = (VOCAB + NB - 1) // NB - 1


def _tc_transpose_kernel(t0_ref, t1_ref, t2_ref, t3_ref, o_ref):
    o_ref[:, 0:32] = t0_ref[...].T
    o_ref[:, 32:64] = t1_ref[...].T
    o_ref[:, 64:96] = t2_ref[...].T
    o_ref[:, 96:128] = t3_ref[...].T


@jax.jit
def _tc_transpose(tt):
    in_specs = [
        pl.BlockSpec(
            (E, NB),
            lambda i, k=k: (0, jnp.minimum(i + k * NBLK, LASTBLK)),
        )
        for k in range(4)
    ]
    return pl.pallas_call(
        _tc_transpose_kernel,
        grid=(NBLK,),
        in_specs=in_specs,
        out_specs=pl.BlockSpec((NB, 128), lambda i: (i, 0)),
        out_shape=jax.ShapeDtypeStruct((QPAD, 128), jnp.float32),
    )(tt, tt, tt, tt)


def _tc_matmul_kernel(p_ref, wt_ref, b_ref, o_ref):
    o_ref[...] = (
        jnp.dot(p_ref[...], wt_ref[...], preferred_element_type=jnp.float32)
        + b_ref[...]
    )


@jax.jit
def _tc_matmul(pooled, Wt, b2):
    BLK = 2048
    return pl.pallas_call(
        _tc_matmul_kernel,
        grid=(B // BLK,),
        in_specs=[
            pl.BlockSpec((BLK, E), lambda i: (i, 0)),
            pl.BlockSpec((E, OUT), lambda i: (0, 0)),
            pl.BlockSpec((1, OUT), lambda i: (0, 0)),
        ],
        out_specs=pl.BlockSpec((BLK, OUT), lambda i: (i, 0)),
        out_shape=jax.ShapeDtypeStruct((B, OUT), jnp.float32),
    )(pooled, Wt, b2)


def kernel(x, table, W, b):
    xi = x.astype(jnp.int32)
    x_perm = 4 * (xi % QPAD) + xi // QPAD
    x_flat = x_perm.reshape(-1)
    t_lin = _tc_transpose(table.T).reshape(VPAD, E)
    pooled = _sc_pool(x_flat, t_lin)
    return _tc_matmul(pooled, W.T, b.reshape(1, OUT))


# speedup vs baseline: 5.8937x; 1.4052x over previous
import functools

import jax
import jax.numpy as jnp
from jax import lax
from jax.experimental import pallas as pl
from jax.experimental.pallas import tpu as pltpu
from jax.experimental.pallas import tpu_sc as plsc

B = 16384
L = 50
E = 32
OUT = 64

NC = 2
NS = 16
NW = NC * NS

RW = B // NW
CR = 32
NCH = RW // CR
NI = CR * L
G = 80
NG = NI // G


def _sc_pool_kernel(x_hbm, table_hbm, out_hbm, idx_v, rows_v, pooled_v, sem):
    wid = lax.axis_index("s") * NC + lax.axis_index("c")
    base_row = wid * RW

    def chunk_body(g, carry):
        row0 = base_row + g * CR
        pltpu.sync_copy(x_hbm.at[pl.ds(row0 * L, NI)], idx_v)
        copies = []
        for j in range(NG):
            copies.append(
                pltpu.async_copy(
                    table_hbm.at[idx_v.at[pl.ds(j * G, G)]],
                    rows_v.at[pl.ds(j * G, G)],
                    sem,
                )
            )
        for c in copies:
            c.wait()

        def reduce_body(c, carry2):
            r0 = c * L
            acc0 = rows_v[r0, pl.ds(0, 16)]
            acc1 = rows_v[r0, pl.ds(16, 16)]
            for l in range(1, L):
                acc0 = acc0 + rows_v[r0 + l, pl.ds(0, 16)]
                acc1 = acc1 + rows_v[r0 + l, pl.ds(16, 16)]
            pooled_v[c, pl.ds(0, 16)] = acc0
            pooled_v[c, pl.ds(16, 16)] = acc1
            return carry2

        lax.fori_loop(0, CR, reduce_body, 0, unroll=False)
        pltpu.sync_copy(pooled_v, out_hbm.at[pl.ds(row0, CR)])
        return carry

    lax.fori_loop(0, NCH, chunk_body, 0, unroll=False)


@jax.jit
def _sc_pool(x_flat, table):
    mesh = plsc.VectorSubcoreMesh(core_axis_name="c", subcore_axis_name="s")
    kfn = pl.kernel(
        _sc_pool_kernel,
        mesh=mesh,
        out_type=jax.ShapeDtypeStruct((B, E), jnp.float32),
        scratch_types=[
            pltpu.VMEM((NI,), jnp.int32),
            pltpu.VMEM((NI, E), jnp.float32),
            pltpu.VMEM((CR, E), jnp.float32),
            pltpu.SemaphoreType.DMA,
        ],
        compiler_params=pltpu.CompilerParams(use_tc_tiling_on_sc=False),
    )
    return kfn(x_flat, table)


VOCAB = 1000000
NB = 2048
NBLK = 123
QPAD = NB * NBLK
VPAD = 4 * QPAD
LASTBLK = (VOCAB + NB - 1) // NB - 1


def _tc_transpose_kernel(t0_ref, t1_ref, t2_ref, t3_ref, o_ref):
    tcat = jnp.concatenate(
        [t0_ref[...], t1_ref[...], t2_ref[...], t3_ref[...]], axis=0
    )
    o_ref[...] = tcat.T


@jax.jit
def _tc_transpose(tt):
    in_specs = [
        pl.BlockSpec(
            (E, NB),
            lambda i, k=k: (0, jnp.minimum(i + k * NBLK, LASTBLK)),
        )
        for k in range(4)
    ]
    return pl.pallas_call(
        _tc_transpose_kernel,
        grid=(NBLK,),
        in_specs=in_specs,
        out_specs=pl.BlockSpec((NB, 128), lambda i: (i, 0)),
        out_shape=jax.ShapeDtypeStruct((QPAD, 128), jnp.float32),
    )(tt, tt, tt, tt)


def _tc_matmul_kernel(p_ref, wt_ref, b_ref, o_ref):
    o_ref[...] = (
        jnp.dot(p_ref[...], wt_ref[...], preferred_element_type=jnp.float32)
        + b_ref[...]
    )


@jax.jit
def _tc_matmul(pooled, Wt, b2):
    BLK = 2048
    return pl.pallas_call(
        _tc_matmul_kernel,
        grid=(B // BLK,),
        in_specs=[
            pl.BlockSpec((BLK, E), lambda i: (i, 0)),
            pl.BlockSpec((E, OUT), lambda i: (0, 0)),
            pl.BlockSpec((1, OUT), lambda i: (0, 0)),
        ],
        out_specs=pl.BlockSpec((BLK, OUT), lambda i: (i, 0)),
        out_shape=jax.ShapeDtypeStruct((B, OUT), jnp.float32),
    )(pooled, Wt, b2)


def kernel(x, table, W, b):
    xi = x.astype(jnp.int32)
    x_perm = 4 * (xi % QPAD) + xi // QPAD
    x_flat = x_perm.reshape(-1)
    t_lin = _tc_transpose(table.T).reshape(VPAD, E)
    pooled = _sc_pool(x_flat, t_lin)
    return _tc_matmul(pooled, W.T, b.reshape(1, OUT))


# speedup vs baseline: 6.8329x; 1.1594x over previous
import functools

import jax
import jax.numpy as jnp
from jax import lax
from jax.experimental import pallas as pl
from jax.experimental.pallas import tpu as pltpu
from jax.experimental.pallas import tpu_sc as plsc

B = 16384
L = 50
E = 32
OUT = 64

NC = 2
NS = 16
NW = NC * NS

RW = B // NW
CR = 32
NCH = RW // CR
NI = CR * L
G = 80
NG = NI // G


def _sc_pool_kernel(x_hbm, table_hbm, out_hbm, idx_v, rows_v, pooled_v, sem):
    wid = lax.axis_index("s") * NC + lax.axis_index("c")
    base_row = wid * RW

    def chunk_body(g, carry):
        row0 = base_row + g * CR
        pltpu.sync_copy(x_hbm.at[pl.ds(row0 * L, NI)], idx_v)
        copies = []
        for j in range(NG):
            copies.append(
                pltpu.async_copy(
                    table_hbm.at[idx_v.at[pl.ds(j * G, G)]],
                    rows_v.at[pl.ds(j * G, G)],
                    sem,
                )
            )
        for c in copies:
            c.wait()

        def reduce_body(c, carry2):
            r0 = c * L
            acc0 = rows_v[r0, pl.ds(0, 16)]
            acc1 = rows_v[r0, pl.ds(16, 16)]
            for l in range(1, L):
                acc0 = acc0 + rows_v[r0 + l, pl.ds(0, 16)]
                acc1 = acc1 + rows_v[r0 + l, pl.ds(16, 16)]
            pooled_v[c, pl.ds(0, 16)] = acc0
            pooled_v[c, pl.ds(16, 16)] = acc1
            return carry2

        lax.fori_loop(0, CR, reduce_body, 0, unroll=False)
        pltpu.sync_copy(pooled_v, out_hbm.at[pl.ds(row0, CR)])
        return carry

    lax.fori_loop(0, NCH, chunk_body, 0, unroll=False)


@jax.jit
def _sc_pool(x_flat, table):
    mesh = plsc.VectorSubcoreMesh(core_axis_name="c", subcore_axis_name="s")
    kfn = pl.kernel(
        _sc_pool_kernel,
        mesh=mesh,
        out_type=jax.ShapeDtypeStruct((B, E), jnp.float32),
        scratch_types=[
            pltpu.VMEM((NI,), jnp.int32),
            pltpu.VMEM((NI, E), jnp.float32),
            pltpu.VMEM((CR, E), jnp.float32),
            pltpu.SemaphoreType.DMA,
        ],
        compiler_params=pltpu.CompilerParams(use_tc_tiling_on_sc=False),
    )
    return kfn(x_flat, table)


VOCAB = 1000000
NB = 4096
NBLK = 62
QPAD = NB * NBLK
VPAD = 4 * QPAD
LASTBLK = (VOCAB + NB - 1) // NB - 1


def _tc_transpose_kernel(t0_ref, t1_ref, t2_ref, t3_ref, o_ref):
    tcat = jnp.concatenate(
        [t0_ref[...], t1_ref[...], t2_ref[...], t3_ref[...]], axis=0
    )
    o_ref[...] = tcat.T


@jax.jit
def _tc_transpose(tt):
    in_specs = [
        pl.BlockSpec(
            (E, NB),
            lambda i, k=k: (0, jnp.minimum(i + k * NBLK, LASTBLK)),
        )
        for k in range(4)
    ]
    return pl.pallas_call(
        _tc_transpose_kernel,
        grid=(NBLK,),
        in_specs=in_specs,
        out_specs=pl.BlockSpec((NB, 128), lambda i: (i, 0)),
        out_shape=jax.ShapeDtypeStruct((QPAD, 128), jnp.float32),
    )(tt, tt, tt, tt)


def _tc_matmul_kernel(p_ref, wt_ref, b_ref, o_ref):
    o_ref[...] = (
        jnp.dot(p_ref[...], wt_ref[...], preferred_element_type=jnp.float32)
        + b_ref[...]
    )


@jax.jit
def _tc_matmul(pooled, Wt, b2):
    BLK = 2048
    return pl.pallas_call(
        _tc_matmul_kernel,
        grid=(B // BLK,),
        in_specs=[
            pl.BlockSpec((BLK, E), lambda i: (i, 0)),
            pl.BlockSpec((E, OUT), lambda i: (0, 0)),
            pl.BlockSpec((1, OUT), lambda i: (0, 0)),
        ],
        out_specs=pl.BlockSpec((BLK, OUT), lambda i: (i, 0)),
        out_shape=jax.ShapeDtypeStruct((B, OUT), jnp.float32),
    )(pooled, Wt, b2)


def kernel(x, table, W, b):
    xi = x.astype(jnp.int32)
    x_perm = 4 * (xi % QPAD) + xi // QPAD
    x_flat = x_perm.reshape(-1)
    t_lin = _tc_transpose(table.T).reshape(VPAD, E)
    pooled = _sc_pool(x_flat, t_lin)
    return _tc_matmul(pooled, W.T, b.reshape(1, OUT))


# speedup vs baseline: 7.2523x; 1.0614x over previous
import functools

import jax
import jax.numpy as jnp
from jax import lax
from jax.experimental import pallas as pl
from jax.experimental.pallas import tpu as pltpu
from jax.experimental.pallas import tpu_sc as plsc

B = 16384
L = 50
E = 32
OUT = 64

NC = 2
NS = 16
NW = NC * NS

RW = B // NW
CR = 32
NCH = RW // CR
NI = CR * L
G = 80
NG = NI // G


def _sc_pool_kernel(x_hbm, table_hbm, out_hbm, idx_v, rows_v, pooled_v, sem):
    wid = lax.axis_index("s") * NC + lax.axis_index("c")
    base_row = wid * RW

    def chunk_body(g, carry):
        row0 = base_row + g * CR
        pltpu.sync_copy(x_hbm.at[pl.ds(row0 * L, NI)], idx_v)
        copies = []
        for j in range(NG):
            copies.append(
                pltpu.async_copy(
                    table_hbm.at[idx_v.at[pl.ds(j * G, G)]],
                    rows_v.at[pl.ds(j * G, G)],
                    sem,
                )
            )
        for c in copies:
            c.wait()

        def reduce_body(c, carry2):
            r0 = c * L
            acc0 = rows_v[r0, pl.ds(0, 16)]
            acc1 = rows_v[r0, pl.ds(16, 16)]
            for l in range(1, L):
                acc0 = acc0 + rows_v[r0 + l, pl.ds(0, 16)]
                acc1 = acc1 + rows_v[r0 + l, pl.ds(16, 16)]
            pooled_v[c, pl.ds(0, 16)] = acc0
            pooled_v[c, pl.ds(16, 16)] = acc1
            return carry2

        lax.fori_loop(0, CR, reduce_body, 0, unroll=False)
        pltpu.sync_copy(pooled_v, out_hbm.at[pl.ds(row0, CR)])
        return carry

    lax.fori_loop(0, NCH, chunk_body, 0, unroll=False)


@jax.jit
def _sc_pool(x_flat, table):
    mesh = plsc.VectorSubcoreMesh(core_axis_name="c", subcore_axis_name="s")
    kfn = pl.kernel(
        _sc_pool_kernel,
        mesh=mesh,
        out_type=jax.ShapeDtypeStruct((B, E), jnp.float32),
        scratch_types=[
            pltpu.VMEM((NI,), jnp.int32),
            pltpu.VMEM((NI, E), jnp.float32),
            pltpu.VMEM((CR, E), jnp.float32),
            pltpu.SemaphoreType.DMA,
        ],
        compiler_params=pltpu.CompilerParams(use_tc_tiling_on_sc=False),
    )
    return kfn(x_flat, table)


VOCAB = 1000000
NB = 8192
NBLK = 31
QPAD = NB * NBLK
VPAD = 4 * QPAD
LASTBLK = (VOCAB + NB - 1) // NB - 1


def _tc_transpose_kernel(t0_ref, t1_ref, t2_ref, t3_ref, o_ref):
    tcat = jnp.concatenate(
        [t0_ref[...], t1_ref[...], t2_ref[...], t3_ref[...]], axis=0
    )
    o_ref[...] = tcat.T


@jax.jit
def _tc_transpose(tt):
    in_specs = [
        pl.BlockSpec(
            (E, NB),
            lambda i, k=k: (0, jnp.minimum(i + k * NBLK, LASTBLK)),
        )
        for k in range(4)
    ]
    return pl.pallas_call(
        _tc_transpose_kernel,
        grid=(NBLK,),
        in_specs=in_specs,
        out_specs=pl.BlockSpec((NB, 128), lambda i: (i, 0)),
        out_shape=jax.ShapeDtypeStruct((QPAD, 128), jnp.float32),
    )(tt, tt, tt, tt)


def _tc_matmul_kernel(p_ref, wt_ref, b_ref, o_ref):
    o_ref[...] = (
        jnp.dot(p_ref[...], wt_ref[...], preferred_element_type=jnp.float32)
        + b_ref[...]
    )


@jax.jit
def _tc_matmul(pooled, Wt, b2):
    BLK = 2048
    return pl.pallas_call(
        _tc_matmul_kernel,
        grid=(B // BLK,),
        in_specs=[
            pl.BlockSpec((BLK, E), lambda i: (i, 0)),
            pl.BlockSpec((E, OUT), lambda i: (0, 0)),
            pl.BlockSpec((1, OUT), lambda i: (0, 0)),
        ],
        out_specs=pl.BlockSpec((BLK, OUT), lambda i: (i, 0)),
        out_shape=jax.ShapeDtypeStruct((B, OUT), jnp.float32),
    )(pooled, Wt, b2)


def kernel(x, table, W, b):
    xi = x.astype(jnp.int32)
    x_perm = 4 * (xi % QPAD) + xi // QPAD
    x_flat = x_perm.reshape(-1)
    t_lin = _tc_transpose(table.T).reshape(VPAD, E)
    pooled = _sc_pool(x_flat, t_lin)
    return _tc_matmul(pooled, W.T, b.reshape(1, OUT))


# speedup vs baseline: 8.6910x; 1.1984x over previous
import functools

import jax
import jax.numpy as jnp
from jax import lax
from jax.experimental import pallas as pl
from jax.experimental.pallas import tpu as pltpu
from jax.experimental.pallas import tpu_sc as plsc

B = 16384
L = 50
E = 32
OUT = 64

NC = 2
NS = 16
NW = NC * NS

RW = B // NW
CR = 32
NCH = RW // CR
NI = CR * L
G = 80
NG = NI // G


def _sc_pool_kernel(
    x_hbm, table_hbm, out_hbm,
    idx_v, rows_v, pooled_v,
    gsem0, gsem1, isem0, isem1,
):
    wid = lax.axis_index("s") * NC + lax.axis_index("c")
    base_row = wid * RW
    gsems = (gsem0, gsem1)
    isems = (isem0, isem1)

    def stage_idx(g):
        row0 = base_row + g * CR
        return pltpu.async_copy(
            x_hbm.at[pl.ds(row0 * L, NI)], idx_v.at[g % 2], isems[g % 2]
        )

    def fire_gathers(g):
        copies = []
        for j in range(NG):
            copies.append(
                pltpu.async_copy(
                    table_hbm.at[idx_v.at[g % 2].at[pl.ds(j * G, G)]],
                    rows_v.at[g % 2].at[pl.ds(j * G, G)],
                    gsems[g % 2],
                )
            )
        return copies

    def reduce_chunk(g):
        par = g % 2

        def reduce_body(c, carry2):
            r0 = c * L
            acc0a = rows_v[par, r0 + 0, pl.ds(0, 16)]
            acc1a = rows_v[par, r0 + 0, pl.ds(16, 16)]
            acc0b = rows_v[par, r0 + 1, pl.ds(0, 16)]
            acc1b = rows_v[par, r0 + 1, pl.ds(16, 16)]
            for l in range(2, L, 2):
                acc0a = acc0a + rows_v[par, r0 + l, pl.ds(0, 16)]
                acc1a = acc1a + rows_v[par, r0 + l, pl.ds(16, 16)]
            for l in range(3, L, 2):
                acc0b = acc0b + rows_v[par, r0 + l, pl.ds(0, 16)]
                acc1b = acc1b + rows_v[par, r0 + l, pl.ds(16, 16)]
            pooled_v[par, c, pl.ds(0, 16)] = acc0a + acc0b
            pooled_v[par, c, pl.ds(16, 16)] = acc1a + acc1b
            return carry2

        lax.fori_loop(0, CR, reduce_body, 0, unroll=False)
        row0 = base_row + g * CR
        pltpu.sync_copy(pooled_v.at[par], out_hbm.at[pl.ds(row0, CR)])

    stage_idx(0).wait()
    pending = fire_gathers(0)
    idx_next = stage_idx(1)
    for g in range(NCH):
        if g + 1 < NCH:
            idx_next.wait()
            nxt = fire_gathers(g + 1)
            if g + 2 < NCH:
                idx_next = stage_idx(g + 2)
        for c in pending:
            c.wait()
        reduce_chunk(g)
        if g + 1 < NCH:
            pending = nxt


@jax.jit
def _sc_pool(x_flat, table):
    mesh = plsc.VectorSubcoreMesh(core_axis_name="c", subcore_axis_name="s")
    kfn = pl.kernel(
        _sc_pool_kernel,
        mesh=mesh,
        out_type=jax.ShapeDtypeStruct((B, E), jnp.float32),
        scratch_types=[
            pltpu.VMEM((2, NI), jnp.int32),
            pltpu.VMEM((2, NI, E), jnp.float32),
            pltpu.VMEM((2, CR, E), jnp.float32),
            pltpu.SemaphoreType.DMA,
            pltpu.SemaphoreType.DMA,
            pltpu.SemaphoreType.DMA,
            pltpu.SemaphoreType.DMA,
        ],
        compiler_params=pltpu.CompilerParams(use_tc_tiling_on_sc=False),
    )
    return kfn(x_flat, table)


VOCAB = 1000000
NB = 8192
NBLK = 31
QPAD = NB * NBLK
VPAD = 4 * QPAD
LASTBLK = (VOCAB + NB - 1) // NB - 1


def _tc_transpose_kernel(t0_ref, t1_ref, t2_ref, t3_ref, o_ref):
    tcat = jnp.concatenate(
        [t0_ref[...], t1_ref[...], t2_ref[...], t3_ref[...]], axis=0
    )
    o_ref[...] = tcat.T


@jax.jit
def _tc_transpose(tt):
    in_specs = [
        pl.BlockSpec(
            (E, NB),
            lambda i, k=k: (0, jnp.minimum(i + k * NBLK, LASTBLK)),
        )
        for k in range(4)
    ]
    return pl.pallas_call(
        _tc_transpose_kernel,
        grid=(NBLK,),
        in_specs=in_specs,
        out_specs=pl.BlockSpec((NB, 128), lambda i: (i, 0)),
        out_shape=jax.ShapeDtypeStruct((QPAD, 128), jnp.float32),
    )(tt, tt, tt, tt)


def _tc_matmul_kernel(p_ref, wt_ref, b_ref, o_ref):
    o_ref[...] = (
        jnp.dot(p_ref[...], wt_ref[...], preferred_element_type=jnp.float32)
        + b_ref[...]
    )


@jax.jit
def _tc_matmul(pooled, Wt, b2):
    BLK = 2048
    return pl.pallas_call(
        _tc_matmul_kernel,
        grid=(B // BLK,),
        in_specs=[
            pl.BlockSpec((BLK, E), lambda i: (i, 0)),
            pl.BlockSpec((E, OUT), lambda i: (0, 0)),
            pl.BlockSpec((1, OUT), lambda i: (0, 0)),
        ],
        out_specs=pl.BlockSpec((BLK, OUT), lambda i: (i, 0)),
        out_shape=jax.ShapeDtypeStruct((B, OUT), jnp.float32),
    )(pooled, Wt, b2)


def kernel(x, table, W, b):
    xi = x.astype(jnp.int32)
    x_perm = 4 * (xi % QPAD) + xi // QPAD
    x_flat = x_perm.reshape(-1)
    t_lin = _tc_transpose(table.T).reshape(VPAD, E)
    pooled = _sc_pool(x_flat, t_lin)
    return _tc_matmul(pooled, W.T, b.reshape(1, OUT))


# speedup vs baseline: 8.7167x; 1.0030x over previous
import functools

import jax
import jax.numpy as jnp
from jax import lax
from jax.experimental import pallas as pl
from jax.experimental.pallas import tpu as pltpu
from jax.experimental.pallas import tpu_sc as plsc

B = 16384
L = 50
E = 32
OUT = 64

NC = 2
NS = 16
NW = NC * NS

RW = B // NW
CR = 32
NCH = RW // CR
NI = CR * L
G = 80
NG = NI // G


def _sc_pool_kernel(
    x_hbm, table_hbm, out_hbm,
    idx_v, rows_v, pooled_v,
    gsem0, gsem1, isem0, isem1,
):
    wid = lax.axis_index("s") * NC + lax.axis_index("c")
    base_row = wid * RW
    gsems = (gsem0, gsem1)
    isems = (isem0, isem1)

    def stage_idx(g):
        row0 = base_row + g * CR
        return pltpu.async_copy(
            x_hbm.at[pl.ds(row0 * L, NI)], idx_v.at[g % 2], isems[g % 2]
        )

    def fire_gathers(g):
        copies = []
        for j in range(NG):
            copies.append(
                pltpu.async_copy(
                    table_hbm.at[idx_v.at[g % 2].at[pl.ds(j * G, G)]],
                    rows_v.at[g % 2].at[pl.ds(j * G, G)],
                    gsems[g % 2],
                )
            )
        return copies

    def reduce_chunk(g):
        par = g % 2

        def reduce_body(c, carry2):
            r0 = c * L
            acc0a = rows_v[par, r0 + 0, pl.ds(0, 16)]
            acc1a = rows_v[par, r0 + 0, pl.ds(16, 16)]
            acc0b = rows_v[par, r0 + 1, pl.ds(0, 16)]
            acc1b = rows_v[par, r0 + 1, pl.ds(16, 16)]
            for l in range(2, L, 2):
                acc0a = acc0a + rows_v[par, r0 + l, pl.ds(0, 16)]
                acc1a = acc1a + rows_v[par, r0 + l, pl.ds(16, 16)]
            for l in range(3, L, 2):
                acc0b = acc0b + rows_v[par, r0 + l, pl.ds(0, 16)]
                acc1b = acc1b + rows_v[par, r0 + l, pl.ds(16, 16)]
            pooled_v[par, c, pl.ds(0, 16)] = acc0a + acc0b
            pooled_v[par, c, pl.ds(16, 16)] = acc1a + acc1b
            return carry2

        lax.fori_loop(0, CR, reduce_body, 0, unroll=2)
        row0 = base_row + g * CR
        pltpu.sync_copy(pooled_v.at[par], out_hbm.at[pl.ds(row0, CR)])

    stage_idx(0).wait()
    pending = fire_gathers(0)
    idx_next = stage_idx(1)
    for g in range(NCH):
        if g + 1 < NCH:
            idx_next.wait()
            nxt = fire_gathers(g + 1)
            if g + 2 < NCH:
                idx_next = stage_idx(g + 2)
        for c in pending:
            c.wait()
        reduce_chunk(g)
        if g + 1 < NCH:
            pending = nxt


@jax.jit
def _sc_pool(x_flat, table):
    mesh = plsc.VectorSubcoreMesh(core_axis_name="c", subcore_axis_name="s")
    kfn = pl.kernel(
        _sc_pool_kernel,
        mesh=mesh,
        out_type=jax.ShapeDtypeStruct((B, E), jnp.float32),
        scratch_types=[
            pltpu.VMEM((2, NI), jnp.int32),
            pltpu.VMEM((2, NI, E), jnp.float32),
            pltpu.VMEM((2, CR, E), jnp.float32),
            pltpu.SemaphoreType.DMA,
            pltpu.SemaphoreType.DMA,
            pltpu.SemaphoreType.DMA,
            pltpu.SemaphoreType.DMA,
        ],
        compiler_params=pltpu.CompilerParams(use_tc_tiling_on_sc=False),
    )
    return kfn(x_flat, table)


VOCAB = 1000000
NB = 16384
NBLK = 16
QPAD = NB * NBLK
VPAD = 4 * QPAD
LASTBLK = (VOCAB + NB - 1) // NB - 1


def _tc_transpose_kernel(t0_ref, t1_ref, t2_ref, t3_ref, o_ref):
    tcat = jnp.concatenate(
        [t0_ref[...], t1_ref[...], t2_ref[...], t3_ref[...]], axis=0
    )
    o_ref[...] = tcat.T


@jax.jit
def _tc_transpose(tt):
    in_specs = [
        pl.BlockSpec(
            (E, NB),
            lambda i, k=k: (0, jnp.minimum(i + k * NBLK, LASTBLK)),
        )
        for k in range(4)
    ]
    return pl.pallas_call(
        _tc_transpose_kernel,
        grid=(NBLK,),
        in_specs=in_specs,
        out_specs=pl.BlockSpec((NB, 128), lambda i: (i, 0)),
        out_shape=jax.ShapeDtypeStruct((QPAD, 128), jnp.float32),
    )(tt, tt, tt, tt)


def _tc_matmul_kernel(p_ref, wt_ref, b_ref, o_ref):
    o_ref[...] = (
        jnp.dot(p_ref[...], wt_ref[...], preferred_element_type=jnp.float32)
        + b_ref[...]
    )


@jax.jit
def _tc_matmul(pooled, Wt, b2):
    BLK = 2048
    return pl.pallas_call(
        _tc_matmul_kernel,
        grid=(B // BLK,),
        in_specs=[
            pl.BlockSpec((BLK, E), lambda i: (i, 0)),
            pl.BlockSpec((E, OUT), lambda i: (0, 0)),
            pl.BlockSpec((1, OUT), lambda i: (0, 0)),
        ],
        out_specs=pl.BlockSpec((BLK, OUT), lambda i: (i, 0)),
        out_shape=jax.ShapeDtypeStruct((B, OUT), jnp.float32),
    )(pooled, Wt, b2)


def kernel(x, table, W, b):
    xi = x.astype(jnp.int32)
    x_perm = 4 * (xi % QPAD) + xi // QPAD
    x_flat = x_perm.reshape(-1)
    t_lin = _tc_transpose(table.T).reshape(VPAD, E)
    pooled = _sc_pool(x_flat, t_lin)
    return _tc_matmul(pooled, W.T, b.reshape(1, OUT))


# speedup vs baseline: 9.0207x; 1.0349x over previous
import functools

import jax
import jax.numpy as jnp
from jax import lax
from jax.experimental import pallas as pl
from jax.experimental.pallas import tpu as pltpu
from jax.experimental.pallas import tpu_sc as plsc

B = 16384
L = 50
E = 32
OUT = 64

NC = 2
NS = 16
NW = NC * NS

RW = B // NW
CR = 32
NCH = RW // CR
NI = CR * L
G = 80
NG = NI // G


def _sc_pool_kernel(
    x_hbm, table_hbm, out_hbm,
    idx_v, rows_v, pooled_v,
    gsem0, gsem1, isem0, isem1,
):
    wid = lax.axis_index("s") * NC + lax.axis_index("c")
    base_row = wid * RW
    gsems = (gsem0, gsem1)
    isems = (isem0, isem1)

    def stage_idx(g):
        row0 = base_row + g * CR
        return pltpu.async_copy(
            x_hbm.at[pl.ds(row0 * L, NI)], idx_v.at[g % 2], isems[g % 2]
        )

    def fire_gathers(g):
        copies = []
        for j in range(NG):
            copies.append(
                pltpu.async_copy(
                    table_hbm.at[idx_v.at[g % 2].at[pl.ds(j * G, G)]],
                    rows_v.at[g % 2].at[pl.ds(j * G, G)],
                    gsems[g % 2],
                )
            )
        return copies

    def reduce_chunk(g):
        par = g % 2

        def reduce_body(c, carry2):
            r0 = c * L
            acc0a = rows_v[par, r0 + 0, pl.ds(0, 16)]
            acc1a = rows_v[par, r0 + 0, pl.ds(16, 16)]
            acc0b = rows_v[par, r0 + 1, pl.ds(0, 16)]
            acc1b = rows_v[par, r0 + 1, pl.ds(16, 16)]
            for l in range(2, L, 2):
                acc0a = acc0a + rows_v[par, r0 + l, pl.ds(0, 16)]
                acc1a = acc1a + rows_v[par, r0 + l, pl.ds(16, 16)]
            for l in range(3, L, 2):
                acc0b = acc0b + rows_v[par, r0 + l, pl.ds(0, 16)]
                acc1b = acc1b + rows_v[par, r0 + l, pl.ds(16, 16)]
            pooled_v[par, c, pl.ds(0, 16)] = acc0a + acc0b
            pooled_v[par, c, pl.ds(16, 16)] = acc1a + acc1b
            return carry2

        lax.fori_loop(0, CR, reduce_body, 0, unroll=2)
        row0 = base_row + g * CR
        pltpu.sync_copy(pooled_v.at[par], out_hbm.at[pl.ds(row0, CR)])

    stage_idx(0).wait()
    pending = fire_gathers(0)
    idx_next = stage_idx(1)
    for g in range(NCH):
        if g + 1 < NCH:
            idx_next.wait()
            nxt = fire_gathers(g + 1)
            if g + 2 < NCH:
                idx_next = stage_idx(g + 2)
        for c in pending:
            c.wait()
        reduce_chunk(g)
        if g + 1 < NCH:
            pending = nxt


@jax.jit
def _sc_pool(x_flat, table):
    mesh = plsc.VectorSubcoreMesh(core_axis_name="c", subcore_axis_name="s")
    kfn = pl.kernel(
        _sc_pool_kernel,
        mesh=mesh,
        out_type=jax.ShapeDtypeStruct((B, E), jnp.float32),
        scratch_types=[
            pltpu.VMEM((2, NI), jnp.int32),
            pltpu.VMEM((2, NI, E), jnp.float32),
            pltpu.VMEM((2, CR, E), jnp.float32),
            pltpu.SemaphoreType.DMA,
            pltpu.SemaphoreType.DMA,
            pltpu.SemaphoreType.DMA,
            pltpu.SemaphoreType.DMA,
        ],
        compiler_params=pltpu.CompilerParams(use_tc_tiling_on_sc=False),
    )
    return kfn(x_flat, table)


VOCAB = 1000000
NB = 16384
NBLK = 16
QPAD = NB * NBLK
VPAD = 4 * QPAD
LASTBLK = (VOCAB + NB - 1) // NB - 1


def _tc_transpose_kernel(t0_ref, t1_ref, t2_ref, t3_ref, o_ref):
    tcat = jnp.concatenate(
        [t0_ref[...], t1_ref[...], t2_ref[...], t3_ref[...]], axis=0
    )
    o_ref[...] = tcat.T


@jax.jit
def _tc_transpose(tt):
    in_specs = [
        pl.BlockSpec(
            (E, NB),
            lambda i, k=k: (0, jnp.minimum(i + k * NBLK, LASTBLK)),
        )
        for k in range(4)
    ]
    return pl.pallas_call(
        _tc_transpose_kernel,
        grid=(NBLK,),
        in_specs=in_specs,
        out_specs=pl.BlockSpec((NB, 128), lambda i: (i, 0)),
        out_shape=jax.ShapeDtypeStruct((QPAD, 128), jnp.float32),
    )(tt, tt, tt, tt)


def _tc_matmul_kernel(p_ref, w_ref, b_ref, o_ref):
    o_ref[...] = (
        jax.lax.dot_general(
            w_ref[...], p_ref[...],
            dimension_numbers=(((1,), (1,)), ((), ())),
            preferred_element_type=jnp.float32,
        )
        + b_ref[...]
    )


@jax.jit
def _tc_matmul(pooled, W, b2):
    BLK = 2048
    return pl.pallas_call(
        _tc_matmul_kernel,
        grid=(B // BLK,),
        in_specs=[
            pl.BlockSpec((BLK, E), lambda i: (i, 0)),
            pl.BlockSpec((OUT, E), lambda i: (0, 0)),
            pl.BlockSpec((OUT, 1), lambda i: (0, 0)),
        ],
        out_specs=pl.BlockSpec((OUT, BLK), lambda i: (0, i)),
        out_shape=jax.ShapeDtypeStruct((OUT, B), jnp.float32),
    )(pooled, W, b2)


def kernel(x, table, W, b):
    x_flat0 = x.astype(jnp.int32).reshape(-1)
    x_flat = 4 * (x_flat0 % QPAD) + x_flat0 // QPAD
    t_lin = _tc_transpose(table.T).reshape(VPAD, E)
    pooled = _sc_pool(x_flat, t_lin)
    return _tc_matmul(pooled, W, b.reshape(OUT, 1)).T


# speedup vs baseline: 9.0324x; 1.0013x over previous
import jax
import jax.numpy as jnp
from jax import lax
from jax.experimental import pallas as pl
from jax.experimental.pallas import tpu as pltpu
from jax.experimental.pallas import tpu_sc as plsc

B = 16384
L = 50
E = 32
OUT = 64

NC = 2
NS = 16
NW = NC * NS

RW = B // NW
CR = 32
NCH = RW // CR
NI = CR * L
G = 80
NG = NI // G


def _sc_pool_kernel(
    x_hbm, table_hbm, out_hbm,
    idx_v, rows_v, pooled_v,
    gsem0, gsem1, isem0, isem1,
):
    wid = lax.axis_index("s") * NC + lax.axis_index("c")
    base_row = wid * RW
    gsems = (gsem0, gsem1)
    isems = (isem0, isem1)

    def stage_idx(g):
        row0 = base_row + g * CR
        return pltpu.async_copy(
            x_hbm.at[pl.ds(row0 * L, NI)], idx_v.at[g % 2], isems[g % 2]
        )

    def fire_gathers(g):
        copies = []
        for j in range(NG):
            copies.append(
                pltpu.async_copy(
                    table_hbm.at[idx_v.at[g % 2].at[pl.ds(j * G, G)]],
                    rows_v.at[g % 2].at[pl.ds(j * G, G)],
                    gsems[g % 2],
                )
            )
        return copies

    def reduce_chunk(g):
        par = g % 2

        def reduce_body(c, carry2):
            r0 = c * L
            acc0a = rows_v[par, r0 + 0, pl.ds(0, 16)]
            acc1a = rows_v[par, r0 + 0, pl.ds(16, 16)]
            acc0b = rows_v[par, r0 + 1, pl.ds(0, 16)]
            acc1b = rows_v[par, r0 + 1, pl.ds(16, 16)]
            for l in range(2, L, 2):
                acc0a = acc0a + rows_v[par, r0 + l, pl.ds(0, 16)]
                acc1a = acc1a + rows_v[par, r0 + l, pl.ds(16, 16)]
            for l in range(3, L, 2):
                acc0b = acc0b + rows_v[par, r0 + l, pl.ds(0, 16)]
                acc1b = acc1b + rows_v[par, r0 + l, pl.ds(16, 16)]
            pooled_v[par, c, pl.ds(0, 16)] = acc0a + acc0b
            pooled_v[par, c, pl.ds(16, 16)] = acc1a + acc1b
            return carry2

        lax.fori_loop(0, CR, reduce_body, 0, unroll=2)
        row0 = base_row + g * CR
        pltpu.sync_copy(pooled_v.at[par], out_hbm.at[pl.ds(row0, CR)])

    stage_idx(0).wait()
    pending = fire_gathers(0)
    idx_next = stage_idx(1)
    for g in range(NCH):
        if g + 1 < NCH:
            idx_next.wait()
            nxt = fire_gathers(g + 1)
            if g + 2 < NCH:
                idx_next = stage_idx(g + 2)
        for c in pending:
            c.wait()
        reduce_chunk(g)
        if g + 1 < NCH:
            pending = nxt


@jax.jit
def _sc_pool(x_flat, table):
    mesh = plsc.VectorSubcoreMesh(core_axis_name="c", subcore_axis_name="s")
    kfn = pl.kernel(
        _sc_pool_kernel,
        mesh=mesh,
        out_type=jax.ShapeDtypeStruct((B, E), jnp.float32),
        scratch_types=[
            pltpu.VMEM((2, NI), jnp.int32),
            pltpu.VMEM((2, NI, E), jnp.float32),
            pltpu.VMEM((2, CR, E), jnp.float32),
            pltpu.SemaphoreType.DMA,
            pltpu.SemaphoreType.DMA,
            pltpu.SemaphoreType.DMA,
            pltpu.SemaphoreType.DMA,
        ],
        compiler_params=pltpu.CompilerParams(use_tc_tiling_on_sc=False),
    )
    return kfn(x_flat, table)


VOCAB = 1000000
NB = 16384
NBLK = 16
QPAD = NB * NBLK
VPAD = 4 * QPAD
LASTBLK = (VOCAB + NB - 1) // NB - 1


def _tc_transpose_kernel(t0_ref, t1_ref, t2_ref, t3_ref, o_ref):
    tcat = jnp.concatenate(
        [t0_ref[...], t1_ref[...], t2_ref[...], t3_ref[...]], axis=0
    )
    o_ref[...] = tcat.T


@jax.jit
def _tc_transpose(tt):
    in_specs = [
        pl.BlockSpec(
            (E, NB),
            lambda i, k=k: (0, jnp.minimum(i + k * NBLK, LASTBLK)),
        )
        for k in range(4)
    ]
    return pl.pallas_call(
        _tc_transpose_kernel,
        grid=(NBLK,),
        in_specs=in_specs,
        out_specs=pl.BlockSpec((NB, 128), lambda i: (i, 0)),
        out_shape=jax.ShapeDtypeStruct((QPAD, 128), jnp.float32),
    )(tt, tt, tt, tt)


def _tc_matmul_kernel(p_ref, w_ref, b_ref, o_ref):
    o_ref[...] = (
        jax.lax.dot_general(
            w_ref[...], p_ref[...],
            dimension_numbers=(((1,), (1,)), ((), ())),
            preferred_element_type=jnp.float32,
        )
        + b_ref[...]
    )


@jax.jit
def _tc_matmul(pooled, W, b2):
    BLK = 2048
    return pl.pallas_call(
        _tc_matmul_kernel,
        grid=(B // BLK,),
        in_specs=[
            pl.BlockSpec((BLK, E), lambda i: (i, 0)),
            pl.BlockSpec((OUT, E), lambda i: (0, 0)),
            pl.BlockSpec((OUT, 1), lambda i: (0, 0)),
        ],
        out_specs=pl.BlockSpec((OUT, BLK), lambda i: (0, i)),
        out_shape=jax.ShapeDtypeStruct((OUT, B), jnp.float32),
    )(pooled, W, b2)


def kernel(x, table, W, b):
    x_flat0 = x.astype(jnp.int32).reshape(-1)
    x_flat = 4 * (x_flat0 % QPAD) + x_flat0 // QPAD
    t_lin = _tc_transpose(table.T).reshape(VPAD, E)
    pooled = _sc_pool(x_flat, t_lin)
    return _tc_matmul(pooled, W, b.reshape(OUT, 1)).T
